# baseline (speedup 1.0000x reference)
import dataclasses
import functools

import jax
import jax.numpy as jnp
from jax import lax
from jax.experimental import pallas as pl
from jax.experimental.pallas import tpu as pltpu
from jax.experimental.pallas import tpu_sc as plsc

NC, NS, L = 2, 16, 16
NW = NC * NS
CHUNK = 128
HALF = CHUNK // 2
HGROUPS = HALF // L
DIMS = ((0, 1), (0, 2), (1, 2))


def _compiler_params():
    cp = pltpu.CompilerParams(use_tc_tiling_on_sc=False)
    if "needs_layout_passes" in pltpu.CompilerParams.__dataclass_fields__:
        cp = dataclasses.replace(cp, needs_layout_passes=False)
    return cp


def _make_sc_sampler(B, C, H, W, N):
    assert C == 4 * L
    n_per_tile = N // NW
    cpb = n_per_tile // CHUNK
    n_chunks = B * cpb
    mesh = plsc.VectorSubcoreMesh(
        core_axis_name="c", subcore_axis_name="s", num_cores=NC, num_subcores=NS
    )
    f32, i32 = jnp.float32, jnp.int32

    C2 = C // 2
    QW = 4 * C2
    scratch = (
        [pltpu.VMEM((HALF,), i32) for _ in range(12)]
        + [pltpu.VMEM((HALF,), f32) for _ in range(48)]
        + [pltpu.VMEM((HALF, QW), f32) for _ in range(6)]
        + [pltpu.VMEM((CHUNK,), f32) for _ in range(6)]
        + [pltpu.VMEM((HALF, 3 * C), f32) for _ in range(2)]
        + [pltpu.SemaphoreType.DMA for _ in range(4)]
    )

    @functools.partial(
        pl.kernel,
        out_type=jax.ShapeDtypeStruct((B, N, 3 * C), f32),
        mesh=mesh,
        compiler_params=_compiler_params(),
        scratch_types=scratch,
    )
    def sampler(t_xy, t_xz, t_yz, xyz1d, out, *refs):
        def IDX(s, h, p):
            return refs[6 * s + 3 * h + p]

        def WGT(s, h, p):
            return refs[12 + 24 * s + 12 * h + 4 * p : 12 + 24 * s + 12 * h + 4 * p + 4]

        def ROWS(h, p):
            return refs[60 + 3 * h + p]

        def CRD(r, d):
            return refs[66 + 3 * r + d]

        outbuf = refs[72:74]
        sem_g = [refs[74], refs[75]]
        sem_c = refs[76]
        sem_o = refs[77]

        wid = lax.axis_index("c") * NS + lax.axis_index("s")
        iota = lax.iota(i32, L)
        tables = (t_xy, t_xz, t_yz)

        def coord_offset(c, d):
            b = c // cpb
            k = c % cpb
            return (b * 3 + d) * N + wid * n_per_tile + k * CHUNK

        def fire_coords(c, ring):
            for d in range(3):
                pltpu.async_copy(
                    xyz1d.at[pl.ds(coord_offset(c, d), CHUNK)], CRD(ring, d),
                    sem_c,
                )

        def wait_coords(ring):
            for d in range(3):
                pltpu.make_async_copy(
                    xyz1d.at[pl.ds(0, CHUNK)], CRD(ring, d), sem_c
                ).wait()

        def compute_idx(c, sset, ring):
            row_base = (c // cpb) * (H * W)
            for h in range(2):
                for p, (d0, d1) in enumerate(DIMS):
                    ib = IDX(sset, h, p)
                    w00, w01, w10, w11 = WGT(sset, h, p)
                    for g in range(HGROUPS):
                        sg = pl.ds(h * HALF + g * L, L)
                        so = pl.ds(g * L, L)
                        px = (CRD(ring, d0)[sg] + 1.0) * 0.5 * (W - 1)
                        py = (CRD(ring, d1)[sg] + 1.0) * 0.5 * (H - 1)
                        x0 = jnp.clip(px.astype(i32), 0, W - 2)
                        y0 = jnp.clip(py.astype(i32), 0, H - 2)
                        wx1 = px - x0.astype(f32)
                        wy1 = py - y0.astype(f32)
                        ib[so] = row_base + y0 * W + x0
                        w00[so] = (1.0 - wx1) * (1.0 - wy1)
                        w01[so] = wx1 * (1.0 - wy1)
                        w10[so] = (1.0 - wx1) * wy1
                        w11[so] = wx1 * wy1

        def fire_half(sset, h):
            for p in range(3):
                pltpu.async_copy(
                    tables[p].at[IDX(sset, h, p)], ROWS(h, p), sem_g[h]
                )

        def drain_half(h):
            for p in range(3):
                pltpu.make_async_copy(
                    t_xy.at[pl.ds(0, HALF)], ROWS(h, p), sem_g[h]
                ).wait()

        def combine_half(sset, h):
            ob = outbuf[h]
            for p in range(3):
                rb = ROWS(h, p)
                w00, w01, w10, w11 = WGT(sset, h, p)

                @pl.loop(0, HGROUPS)
                def _(g):
                    sg = pl.ds(g * L, L)
                    ws = (w00[sg], w01[sg], w10[sg], w11[sg])
                    rows = iota + g * L

                    @plsc.parallel_loop(0, C2, unroll=4)
                    def _(j):
                        cj = jnp.full((L,), 0, i32) + j
                        acc_e = jnp.zeros((L,), f32)
                        acc_o = jnp.zeros((L,), f32)
                        for q, aq in enumerate(ws):
                            vi = lax.bitcast_convert_type(
                                plsc.load_gather(rb, [rows, cj + q * C2]), i32
                            )
                            ve = lax.bitcast_convert_type(
                                lax.shift_left(vi, 16), f32
                            )
                            vo = lax.bitcast_convert_type(
                                lax.bitwise_and(
                                    vi, jnp.int32(-65536)
                                ),
                                f32,
                            )
                            acc_e = acc_e + ve * aq
                            acc_o = acc_o + vo * aq
                        plsc.store_scatter(ob, [rows, cj * 2 + p * C], acc_e)
                        plsc.store_scatter(
                            ob, [rows, cj * 2 + (p * C + 1)], acc_o
                        )

        def fire_out(c):
            b = c // cpb
            n0 = (c % cpb) * CHUNK + wid * n_per_tile
            return [
                pltpu.async_copy(
                    outbuf[h], out.at[b, pl.ds(n0 + h * HALF, HALF), :], sem_o
                )
                for h in range(2)
            ]

        fire_coords(0, 0)
        wait_coords(0)
        fire_coords(1, 1)
        compute_idx(0, 0, 0)
        fire_half(0, 0)
        fire_half(0, 1)

        @pl.loop(0, n_chunks, step=2)
        def _(c0):
            for u in range(2):
                c = c0 + u

                @pl.when(c + 1 < n_chunks)
                def _():
                    wait_coords(1 - u)
                    compute_idx(c + 1, 1 - u, 1 - u)

                @pl.when(c + 2 < n_chunks)
                def _():
                    fire_coords(c + 2, u)

                drain_half(0)
                combine_half(u, 0)
                drain_half(1)
                combine_half(u, 1)

                @pl.when(c + 1 < n_chunks)
                def _():
                    fire_half(1 - u, 0)
                    fire_half(1 - u, 1)

                copies = fire_out(c)
                for cp in copies:
                    cp.wait()

    return sampler


def kernel(plane_xy, plane_xz, plane_yz, xyz_norm):
    B, C, H, W = plane_xy.shape
    N = xyz_norm.shape[1]

    def pack(p):
        t = jnp.transpose(p, (0, 2, 3, 1)).astype(jnp.bfloat16)
        z = ((0, 0), (0, 0), (0, 0), (0, 0))
        t01 = jnp.pad(t[:, :, 1:, :], ((0, 0), (0, 0), (0, 1), (0, 0)))
        t10 = jnp.pad(t[:, 1:, :, :], ((0, 0), (0, 1), (0, 0), (0, 0)))
        t11 = jnp.pad(t[:, 1:, 1:, :], ((0, 0), (0, 1), (0, 1), (0, 0)))
        q = jnp.concatenate([t, t01, t10, t11], axis=-1)
        return jax.lax.bitcast_convert_type(
            q.reshape(B * H * W, 2 * C, 2), jnp.float32
        )

    xyz1d = jnp.transpose(xyz_norm, (0, 2, 1)).reshape(B * 3 * N)
    return _make_sc_sampler(B, C, H, W, N)(
        pack(plane_xy), pack(plane_xz), pack(plane_yz), xyz1d
    )


# speedup vs baseline: 1.8126x; 1.8126x over previous
import dataclasses
import functools

import jax
import jax.numpy as jnp
from jax import lax
from jax.experimental import pallas as pl
from jax.experimental.pallas import tpu as pltpu
from jax.experimental.pallas import tpu_sc as plsc

NC, NS, L = 2, 16, 16
NW = NC * NS
CHUNK = 128
HALF = CHUNK // 2
HGROUPS = HALF // L
DIMS = ((0, 1), (0, 2), (1, 2))


def _compiler_params():
    cp = pltpu.CompilerParams(use_tc_tiling_on_sc=False)
    if "needs_layout_passes" in pltpu.CompilerParams.__dataclass_fields__:
        cp = dataclasses.replace(cp, needs_layout_passes=False)
    return cp


def _make_sc_sampler(B, C, H, W, N):
    assert C == 4 * L
    n_per_tile = N // NW
    cpb = n_per_tile // CHUNK
    n_chunks = B * cpb
    mesh = plsc.VectorSubcoreMesh(
        core_axis_name="c", subcore_axis_name="s", num_cores=NC, num_subcores=NS
    )
    f32, i32 = jnp.float32, jnp.int32

    C2 = C // 2
    scratch = (
        [pltpu.VMEM((HALF,), i32) for _ in range(48)]
        + [pltpu.VMEM((HALF,), f32) for _ in range(48)]
        + [pltpu.VMEM((HALF, C2), f32) for _ in range(24)]
        + [pltpu.VMEM((CHUNK,), f32) for _ in range(6)]
        + [pltpu.VMEM((HALF, 3 * C), f32) for _ in range(2)]
        + [pltpu.SemaphoreType.DMA for _ in range(4)]
    )

    @functools.partial(
        pl.kernel,
        out_type=jax.ShapeDtypeStruct((B, N, 3 * C), f32),
        mesh=mesh,
        compiler_params=_compiler_params(),
        scratch_types=scratch,
    )
    def sampler(t_xy, t_xz, t_yz, xyz1d, out, *refs):
        def IDX(s, h, p):
            return refs[24 * s + 12 * h + 4 * p : 24 * s + 12 * h + 4 * p + 4]

        def WGT(s, h, p):
            return refs[48 + 24 * s + 12 * h + 4 * p : 48 + 24 * s + 12 * h + 4 * p + 4]

        def ROWS(h, p):
            return refs[96 + 12 * h + 4 * p : 96 + 12 * h + 4 * p + 4]

        def CRD(r, d):
            return refs[120 + 3 * r + d]

        outbuf = refs[126:128]
        sem_g = [refs[128], refs[129]]
        sem_c = refs[130]
        sem_o = refs[131]

        wid = lax.axis_index("c") * NS + lax.axis_index("s")
        iota = lax.iota(i32, L)
        tables = (t_xy, t_xz, t_yz)

        def coord_offset(c, d):
            b = c // cpb
            k = c % cpb
            return (b * 3 + d) * N + wid * n_per_tile + k * CHUNK

        def fire_coords(c, ring):
            for d in range(3):
                pltpu.async_copy(
                    xyz1d.at[pl.ds(coord_offset(c, d), CHUNK)], CRD(ring, d),
                    sem_c,
                )

        def wait_coords(ring):
            for d in range(3):
                pltpu.make_async_copy(
                    xyz1d.at[pl.ds(0, CHUNK)], CRD(ring, d), sem_c
                ).wait()

        def compute_idx(c, sset, ring):
            row_base = (c // cpb) * (H * W)
            for h in range(2):
                for p, (d0, d1) in enumerate(DIMS):
                    i00, i01, i10, i11 = IDX(sset, h, p)
                    w00, w01, w10, w11 = WGT(sset, h, p)
                    for g in range(HGROUPS):
                        sg = pl.ds(h * HALF + g * L, L)
                        so = pl.ds(g * L, L)
                        px = (CRD(ring, d0)[sg] + 1.0) * 0.5 * (W - 1)
                        py = (CRD(ring, d1)[sg] + 1.0) * 0.5 * (H - 1)
                        x0 = jnp.clip(px.astype(i32), 0, W - 2)
                        y0 = jnp.clip(py.astype(i32), 0, H - 2)
                        wx1 = px - x0.astype(f32)
                        wy1 = py - y0.astype(f32)
                        r = row_base + y0 * W + x0
                        i00[so] = r
                        i01[so] = r + 1
                        i10[so] = r + W
                        i11[so] = r + W + 1
                        w00[so] = (1.0 - wx1) * (1.0 - wy1)
                        w01[so] = wx1 * (1.0 - wy1)
                        w10[so] = (1.0 - wx1) * wy1
                        w11[so] = wx1 * wy1

        def fire_half(sset, h):
            for p in range(3):
                for idx, buf in zip(IDX(sset, h, p), ROWS(h, p)):
                    pltpu.async_copy(tables[p].at[idx], buf, sem_g[h])

        def drain_half(h):
            for p in range(3):
                for buf in ROWS(h, p):
                    pltpu.make_async_copy(
                        t_xy.at[pl.ds(0, HALF)], buf, sem_g[h]
                    ).wait()

        def combine_half(sset, h):
            ob = outbuf[h]
            for p in range(3):
                r00, r01, r10, r11 = ROWS(h, p)
                w00, w01, w10, w11 = WGT(sset, h, p)

                @pl.loop(0, HGROUPS)
                def _(g):
                    sg = pl.ds(g * L, L)
                    ws = (w00[sg], w01[sg], w10[sg], w11[sg])
                    rows = iota + g * L

                    @plsc.parallel_loop(0, C2, unroll=4)
                    def _(j):
                        cj = jnp.full((L,), 0, i32) + j
                        acc_e = jnp.zeros((L,), f32)
                        acc_o = jnp.zeros((L,), f32)
                        for rq, aq in zip((r00, r01, r10, r11), ws):
                            vi = lax.bitcast_convert_type(
                                plsc.load_gather(rq, [rows, cj]), i32
                            )
                            ve = lax.bitcast_convert_type(
                                lax.shift_left(vi, 16), f32
                            )
                            vo = lax.bitcast_convert_type(
                                lax.bitwise_and(
                                    vi, jnp.int32(-65536)
                                ),
                                f32,
                            )
                            acc_e = acc_e + ve * aq
                            acc_o = acc_o + vo * aq
                        plsc.store_scatter(ob, [rows, cj * 2 + p * C], acc_e)
                        plsc.store_scatter(
                            ob, [rows, cj * 2 + (p * C + 1)], acc_o
                        )

        def fire_out(c):
            b = c // cpb
            n0 = (c % cpb) * CHUNK + wid * n_per_tile
            return [
                pltpu.async_copy(
                    outbuf[h], out.at[b, pl.ds(n0 + h * HALF, HALF), :], sem_o
                )
                for h in range(2)
            ]

        fire_coords(0, 0)
        wait_coords(0)
        fire_coords(1, 1)
        compute_idx(0, 0, 0)
        fire_half(0, 0)
        fire_half(0, 1)

        @pl.loop(0, n_chunks, step=2)
        def _(c0):
            for u in range(2):
                c = c0 + u

                @pl.when(c + 1 < n_chunks)
                def _():
                    wait_coords(1 - u)
                    compute_idx(c + 1, 1 - u, 1 - u)

                @pl.when(c + 2 < n_chunks)
                def _():
                    fire_coords(c + 2, u)

                drain_half(0)
                combine_half(u, 0)

                @pl.when(c + 1 < n_chunks)
                def _():
                    fire_half(1 - u, 0)

                drain_half(1)
                combine_half(u, 1)

                @pl.when(c + 1 < n_chunks)
                def _():
                    fire_half(1 - u, 1)

                copies = fire_out(c)
                for cp in copies:
                    cp.wait()

    return sampler


def kernel(plane_xy, plane_xz, plane_yz, xyz_norm):
    B, C, H, W = plane_xy.shape
    N = xyz_norm.shape[1]

    def pack(p):
        t = jnp.transpose(p, (0, 2, 3, 1)).astype(jnp.bfloat16)
        return jax.lax.bitcast_convert_type(
            t.reshape(B * H * W, C // 2, 2), jnp.float32
        )

    xyz1d = jnp.transpose(xyz_norm, (0, 2, 1)).reshape(B * 3 * N)
    return _make_sc_sampler(B, C, H, W, N)(
        pack(plane_xy), pack(plane_xz), pack(plane_yz), xyz1d
    )
